# baseline (speedup 1.0000x reference)
import functools

import jax
import jax.numpy as jnp
from jax import lax
from jax.experimental import pallas as pl
from jax.experimental.pallas import tpu as pltpu
from jax.experimental.pallas import tpu_sc as plsc


@functools.lru_cache(maxsize=None)
def _make_embed_kernel(V, D, N, S):
    info = plsc.get_sparse_core_info()
    NC, NS, L = info.num_cores, info.num_subcores, info.num_lanes
    NW = NC * NS
    assert N % NW == 0
    n_per_w = N // NW
    G = 128
    KG = 8
    CH = G * KG
    assert n_per_w % CH == 0
    n_ch = n_per_w // CH
    assert D % L == 0

    mesh = plsc.VectorSubcoreMesh(core_axis_name="c", subcore_axis_name="s")

    @functools.partial(
        pl.kernel,
        mesh=mesh,
        compiler_params=pltpu.CompilerParams(use_tc_tiling_on_sc=False),
        out_type=jax.ShapeDtypeStruct((N, D), jnp.float32),
        scratch_types=[
            pltpu.VMEM((KG, G), jnp.int32),
            pltpu.VMEM((CH, D), jnp.float32),
            pltpu.VMEM((S, D), jnp.float32),
            pltpu.SemaphoreType.DMA,
        ],
    )
    def embed(table_hbm, idx_hbm, pos_hbm, out_hbm, idx_v, rows_v, pos_v, sem):
        wid = lax.axis_index("s") * NC + lax.axis_index("c")
        base = wid * n_per_w
        pltpu.sync_copy(pos_hbm, pos_v)

        def chunk_body(c, carry):
            row0 = base + c * CH
            g0 = wid * (n_per_w // G) + c * KG
            pltpu.sync_copy(idx_hbm.at[pl.ds(g0, KG)], idx_v)
            cps = [
                pltpu.async_copy(
                    table_hbm.at[idx_v.at[j]],
                    rows_v.at[pl.ds(j * G, G)],
                    sem,
                )
                for j in range(KG)
            ]
            for cp in cps:
                cp.wait()

            def row_body(r, rcarry):
                s = lax.rem(row0 + r, S)
                for t in range(D // L):
                    pv = pos_v[s, pl.ds(t * L, L)]
                    plsc.addupdate(rows_v.at[r, pl.ds(t * L, L)], pv)
                return rcarry

            lax.fori_loop(0, CH, row_body, 0)
            pltpu.sync_copy(rows_v, out_hbm.at[pl.ds(row0, CH)])
            return carry

        lax.fori_loop(0, n_ch, chunk_body, 0)

    return embed


def kernel(x, token_table, pos_table):
    B, S = x.shape
    V, D = token_table.shape
    N = B * S
    G = 128
    idx = x.reshape(N // G, G).astype(jnp.int32)
    embed = _make_embed_kernel(V, D, N, S)
    out = embed(token_table, idx, pos_table)
    return out.reshape(B, S, D)
